# baseline (speedup 1.0000x reference)
import numpy as np

import jax
import jax.numpy as jnp
from jax import lax
from jax.experimental import pallas as pl
from jax.experimental.pallas import tpu as pltpu

try:
    from jax.experimental.shard_map import shard_map as _shard_map
except ImportError:
    _shard_map = jax.shard_map

P = jax.sharding.PartitionSpec

_NPOINT = 1024
_RADIUS = 0.2
_NSAMPLE = 32
_EPS = 1e-5
_B = 16
_N = 4096
_D = 64

_R2 = _RADIUS ** 2
_NROWS = _B * _NPOINT * _NSAMPLE

_F32 = jnp.float32


_G = 8
_SUB = 8
_LANE = _N // _SUB


def _fps_body(x_ref, y_ref, z_ref, *out_and_scratch):
    out_refs = out_and_scratch[:_G]
    scr_refs = out_and_scratch[_G:]
    x = x_ref[0]
    y = y_ref[0]
    z = z_ref[0]
    nidx = (lax.broadcasted_iota(jnp.int32, (_G, _SUB, _LANE), 1)
            * _LANE
            + lax.broadcasted_iota(jnp.int32, (_G, _SUB, _LANE), 2)
            ).astype(_F32)
    li = lax.broadcasted_iota(jnp.int32, (_G, 1, 128), 2)
    big = jnp.float32(1e10)

    def step(i, carry):
        dists, onehot = carry
        px = jnp.sum(jnp.sum(x * onehot, axis=2, keepdims=True), axis=1,
                     keepdims=True)
        py = jnp.sum(jnp.sum(y * onehot, axis=2, keepdims=True), axis=1,
                     keepdims=True)
        pz = jnp.sum(jnp.sum(z * onehot, axis=2, keepdims=True), axis=1,
                     keepdims=True)
        rows = (jnp.where(li == 0, px, 0.0) + jnp.where(li == 1, py, 0.0)
                + jnp.where(li == 2, pz, 0.0))
        for g in range(_G):
            scr_refs[g][pl.ds(i, 1), :, :] = rows[g:g + 1]
        d = ((x - px) ** 2 + (y - py) ** 2) + (z - pz) ** 2
        dists = jnp.minimum(dists, d)
        m = jnp.max(jnp.max(dists, axis=2, keepdims=True), axis=1,
                    keepdims=True)
        cand = jnp.where(dists == m, nidx, big)
        nxt = jnp.min(jnp.min(cand, axis=2, keepdims=True), axis=1,
                      keepdims=True)
        onehot = jnp.where(nidx == nxt, 1.0, 0.0)
        return dists, onehot

    init = (jnp.full((_G, _SUB, _LANE), big, _F32),
            jnp.where(nidx == 0.0, 1.0, 0.0))
    lax.fori_loop(0, _NPOINT, step, init)
    for g in range(_G):
        out_refs[g][0] = scr_refs[g][...]


def _run_fps(xyz, bl):
    npg = bl // _G
    xr = xyz[:, 0, :].reshape(npg, _G, _SUB, _LANE)
    yr = xyz[:, 1, :].reshape(npg, _G, _SUB, _LANE)
    zr = xyz[:, 2, :].reshape(npg, _G, _SUB, _LANE)
    spec = pl.BlockSpec((1, _G, _SUB, _LANE), lambda c: (c, 0, 0, 0))
    out_spec = pl.BlockSpec((1, _NPOINT, 1, 128), lambda c: (c, 0, 0, 0))
    outs = pl.pallas_call(
        _fps_body,
        grid=(npg,),
        in_specs=[spec, spec, spec],
        out_specs=[out_spec] * _G,
        out_shape=[jax.ShapeDtypeStruct((npg, _NPOINT, 1, 128), _F32)] * _G,
        scratch_shapes=[pltpu.VMEM((_NPOINT, 1, 128), _F32)] * _G,
        compiler_params=pltpu.CompilerParams(
            dimension_semantics=("arbitrary",)),
    )(xr, yr, zr)
    nx = jnp.stack(outs, axis=1)[:, :, :, 0, :].reshape(bl, _NPOINT, 128)
    return nx


def _p0_body(xp_ref, w0t_ref, g1_ref):
    g1_ref[0] = jnp.dot(xp_ref[0], w0t_ref[...],
                        preferred_element_type=_F32).astype(jnp.bfloat16)


def _run_p0(xp, w0t, bl):
    return pl.pallas_call(
        _p0_body,
        grid=(bl,),
        in_specs=[pl.BlockSpec((1, _N, 3 + _D), lambda b: (b, 0, 0)),
                  pl.BlockSpec((3 + _D, _D), lambda b: (0, 0))],
        out_specs=pl.BlockSpec((1, _N, _D), lambda b: (b, 0, 0)),
        out_shape=jax.ShapeDtypeStruct((bl, _N, _D), jnp.bfloat16),
        compiler_params=pltpu.CompilerParams(
            dimension_semantics=("arbitrary",)),
    )(xp, w0t)


_SBLK = 128
_NSB = _NPOINT // _SBLK
_SGRP = 8


def _p1_body(xyz_ref, nx_ref, g1_ref, w0a_ref, b0_ref, z1_ref, st_ref,
             sel_ref):
    x = xyz_ref[0, 0:1, :]
    y = xyz_ref[0, 1:2, :]
    z = xyz_ref[0, 2:3, :]
    xyz3 = xyz_ref[0]
    x2 = (x * x + y * y) + z * z
    wx = w0a_ref[0:1]
    wy = w0a_ref[1:2]
    wz = w0a_ref[2:3]
    b0 = b0_ref[...]
    kcol = lax.broadcasted_iota(
        jnp.int32, (_NSAMPLE, 1), 0).astype(_F32)
    kk3 = lax.broadcasted_iota(
        jnp.int32, (_SGRP, _NSAMPLE, 1), 1).astype(_F32)
    s1 = jnp.zeros((1, _D), _F32)
    s2 = jnp.zeros((1, _D), _F32)
    for i in range(_SBLK // _SGRP):
        c8 = nx_ref[0, i * _SGRP:(i + 1) * _SGRP, :]
        cx = c8[:, 0:1]
        cy = c8[:, 1:2]
        cz = c8[:, 2:3]
        mm = jnp.dot(c8[:, 0:3], xyz3, preferred_element_type=_F32)
        cs2 = (cx * cx + cy * cy) + cz * cz
        d = (-2.0 * mm + cs2) + x2
        maskf = jnp.where(d <= _R2, 1.0, 0.0)
        c = maskf
        sh = 1
        while sh < _N:
            c = c + jnp.concatenate(
                [jnp.zeros((_SGRP, sh), _F32), c[:, :-sh]], axis=1)
            sh *= 2
        cnt = c[:, _N - 1:_N]
        cm = c * maskf
        cmb = cm.astype(jnp.bfloat16)
        kcolb = (kcol + 1.0).astype(jnp.bfloat16)
        one_b = jnp.bfloat16(1.0)
        zero_b = jnp.bfloat16(0.0)
        for s in range(_SGRP):
            sel_ref[s * _NSAMPLE:(s + 1) * _NSAMPLE, :] = jnp.where(
                cmb[s:s + 1, :] == kcolb, one_b, zero_b)
        zres = jnp.dot(sel_ref[...], g1_ref[0],
                       preferred_element_type=_F32)
        z3 = zres.reshape(_SGRP, _NSAMPLE, _D)
        row0 = z3[:, 0:1, :]
        z3 = jnp.where(kk3 < cnt[:, :, None], z3, row0)
        t = (cx * wx + cy * wy + cz * wz) - b0
        z1f = (z3 - t[:, None, :]).reshape(_SGRP * _NSAMPLE, _D)
        z1_ref[0, i * 256:(i + 1) * 256, :] = z1f
        s1 = s1 + jnp.sum(z1f, axis=0, keepdims=True)
        s2 = s2 + jnp.sum(z1f * z1f, axis=0, keepdims=True)
    st_ref[0] = jnp.concatenate(
        [s1, s2, jnp.zeros((6, _D), _F32)], axis=0)


def _run_p1(xyz, nx, g1, w0a, b0v, bl):
    nblk = bl * _NSB
    return pl.pallas_call(
        _p1_body,
        grid=(bl, _NSB),
        in_specs=[
            pl.BlockSpec((1, 3, _N), lambda b, j: (b, 0, 0)),
            pl.BlockSpec((1, _SBLK, 128), lambda b, j: (b * _NSB + j, 0, 0)),
            pl.BlockSpec((1, _N, _D), lambda b, j: (b, 0, 0)),
            pl.BlockSpec((3, _D), lambda b, j: (0, 0)),
            pl.BlockSpec((1, _D), lambda b, j: (0, 0)),
        ],
        out_specs=[
            pl.BlockSpec((1, _SBLK * _NSAMPLE, _D),
                         lambda b, j: (b, j, 0)),
            pl.BlockSpec((1, 8, _D), lambda b, j: (b * _NSB + j, 0, 0)),
        ],
        out_shape=[
            jax.ShapeDtypeStruct((bl, _NPOINT * _NSAMPLE, _D), _F32),
            jax.ShapeDtypeStruct((nblk, 8, _D), _F32),
        ],
        scratch_shapes=[pltpu.VMEM((_SGRP * _NSAMPLE, _N), jnp.bfloat16)],
        compiler_params=pltpu.CompilerParams(
            dimension_semantics=("arbitrary", "arbitrary")),
    )(xyz, nx.reshape(bl * _NSB, _SBLK, 128), g1, w0a, b0v)


def _bn_affine(stv, gvec, bevec):
    s1 = jnp.sum(stv[:, 0, :], axis=0, keepdims=True)
    s2 = jnp.sum(stv[:, 1, :], axis=0, keepdims=True)
    mu = s1 / float(_NROWS)
    var = s2 / float(_NROWS) - mu * mu
    alpha = gvec * lax.rsqrt(var + _EPS)
    beta = bevec - mu * alpha
    return alpha, beta


_RBLK = 4096
_NRB = _NPOINT * _NSAMPLE // _RBLK


def _p2_body(z1_ref, st1_ref, w1t_ref, b1_ref, g1v_ref, be1_ref, st2_ref):
    al1, be1 = _bn_affine(st1_ref[...], g1v_ref[...], be1_ref[...])
    a = jnp.maximum(z1_ref[0] * al1 + be1, 0.0)
    z2 = jnp.dot(a, w1t_ref[...], preferred_element_type=_F32) + b1_ref[...]
    s1 = jnp.sum(z2, axis=0, keepdims=True)
    s2 = jnp.sum(z2 * z2, axis=0, keepdims=True)
    st2_ref[0] = jnp.concatenate(
        [s1, s2, jnp.zeros((6, _D), _F32)], axis=0)


def _run_p2(z1, st1, w1t, b1v, g1v, be1v, bl):
    nblk = bl * _NRB
    return pl.pallas_call(
        _p2_body,
        grid=(bl, _NRB),
        in_specs=[
            pl.BlockSpec((1, _RBLK, _D), lambda b, j: (b, j, 0)),
            pl.BlockSpec(st1.shape, lambda b, j: (0, 0, 0)),
            pl.BlockSpec((_D, _D), lambda b, j: (0, 0)),
            pl.BlockSpec((1, _D), lambda b, j: (0, 0)),
            pl.BlockSpec((1, _D), lambda b, j: (0, 0)),
            pl.BlockSpec((1, _D), lambda b, j: (0, 0)),
        ],
        out_specs=pl.BlockSpec((1, 8, _D), lambda b, j: (b * _NRB + j, 0, 0)),
        out_shape=jax.ShapeDtypeStruct((nblk, 8, _D), _F32),
        compiler_params=pltpu.CompilerParams(
            dimension_semantics=("arbitrary", "arbitrary")),
    )(z1, st1, w1t, b1v, g1v, be1v)


def _p3_body(z1_ref, st1_ref, st2_ref, w1t_ref, b1_ref, gl1_ref, bel1_ref,
             gl2_ref, bel2_ref, w2t_ref, b2_ref, st3_ref):
    al1, be1 = _bn_affine(st1_ref[...], gl1_ref[...], bel1_ref[...])
    a = jnp.maximum(z1_ref[0] * al1 + be1, 0.0)
    z2 = jnp.dot(a, w1t_ref[...], preferred_element_type=_F32) + b1_ref[...]
    al2, be2 = _bn_affine(st2_ref[...], gl2_ref[...], bel2_ref[...])
    a2 = jnp.maximum(z2 * al2 + be2, 0.0)
    z3 = jnp.dot(a2, w2t_ref[...], preferred_element_type=_F32) + b2_ref[...]
    s1 = jnp.sum(z3, axis=0, keepdims=True)
    s2 = jnp.sum(z3 * z3, axis=0, keepdims=True)
    st3_ref[0] = jnp.concatenate(
        [s1, s2, jnp.zeros((6, 2 * _D), _F32)], axis=0)


def _run_p3(z1, st1, st2, w1t, b1v, gl1, bel1, gl2, bel2, w2t, b2v, bl):
    nblk = bl * _NRB
    vec = lambda: pl.BlockSpec((1, _D), lambda b, j: (0, 0))
    return pl.pallas_call(
        _p3_body,
        grid=(bl, _NRB),
        in_specs=[
            pl.BlockSpec((1, _RBLK, _D), lambda b, j: (b, j, 0)),
            pl.BlockSpec(st1.shape, lambda b, j: (0, 0, 0)),
            pl.BlockSpec(st2.shape, lambda b, j: (0, 0, 0)),
            pl.BlockSpec((_D, _D), lambda b, j: (0, 0)),
            vec(), vec(), vec(), vec(), vec(),
            pl.BlockSpec((_D, 2 * _D), lambda b, j: (0, 0)),
            pl.BlockSpec((1, 2 * _D), lambda b, j: (0, 0)),
        ],
        out_specs=pl.BlockSpec((1, 8, 2 * _D),
                               lambda b, j: (b * _NRB + j, 0, 0)),
        out_shape=jax.ShapeDtypeStruct((nblk, 8, 2 * _D), _F32),
        compiler_params=pltpu.CompilerParams(
            dimension_semantics=("arbitrary", "arbitrary")),
    )(z1, st1, st2, w1t, b1v, gl1, bel1, gl2, bel2, w2t, b2v)


def _p4_body(z1_ref, st1_ref, st2_ref, st3_ref, w1t_ref, b1_ref,
             gl1_ref, bel1_ref, gl2_ref, bel2_ref, gl3_ref, bel3_ref,
             w2t_ref, b2_ref, out_ref):
    al1, be1 = _bn_affine(st1_ref[...], gl1_ref[...], bel1_ref[...])
    a = jnp.maximum(z1_ref[0] * al1 + be1, 0.0)
    z2 = jnp.dot(a, w1t_ref[...], preferred_element_type=_F32) + b1_ref[...]
    al2, be2 = _bn_affine(st2_ref[...], gl2_ref[...], bel2_ref[...])
    a2 = jnp.maximum(z2 * al2 + be2, 0.0)
    z3 = jnp.dot(a2, w2t_ref[...], preferred_element_type=_F32) + b2_ref[...]
    al3, be3 = _bn_affine(st3_ref[...], gl3_ref[...], bel3_ref[...])
    a3 = jnp.maximum(z3 * al3 + be3, 0.0)
    pooled = jnp.max(a3.reshape(_RBLK // _NSAMPLE, _NSAMPLE, 2 * _D),
                     axis=1)
    out_ref[0] = pooled


def _run_p4(z1, st1, st2, st3, w1t, b1v, gl1, bel1, gl2, bel2, gl3, bel3,
            w2t, b2v, bl):
    vec = lambda: pl.BlockSpec((1, _D), lambda b, j: (0, 0))
    vec2 = lambda: pl.BlockSpec((1, 2 * _D), lambda b, j: (0, 0))
    return pl.pallas_call(
        _p4_body,
        grid=(bl, _NRB),
        in_specs=[
            pl.BlockSpec((1, _RBLK, _D), lambda b, j: (b, j, 0)),
            pl.BlockSpec(st1.shape, lambda b, j: (0, 0, 0)),
            pl.BlockSpec(st2.shape, lambda b, j: (0, 0, 0)),
            pl.BlockSpec(st3.shape, lambda b, j: (0, 0, 0)),
            pl.BlockSpec((_D, _D), lambda b, j: (0, 0)),
            vec(), vec(), vec(), vec(), vec(),
            vec2(), vec2(),
            pl.BlockSpec((_D, 2 * _D), lambda b, j: (0, 0)),
            vec2(),
        ],
        out_specs=pl.BlockSpec((1, _RBLK // _NSAMPLE, 2 * _D),
                               lambda b, j: (b, j, 0)),
        out_shape=jax.ShapeDtypeStruct((bl, _NPOINT, 2 * _D), _F32),
        compiler_params=pltpu.CompilerParams(
            dimension_semantics=("arbitrary", "arbitrary")),
    )(z1, st1, st2, st3, w1t, b1v, gl1, bel1, gl2, bel2, gl3, bel3,
      w2t, b2v)


def _pipeline(xyz, points, w0, b0, g0, be0, w1, b1, g1, be1, w2, b2, g2,
              be2):
    bl = xyz.shape[0]
    xyz = xyz.astype(_F32)
    points = points.astype(_F32)

    nx = _run_fps(xyz, bl)

    xp = jnp.concatenate(
        [xyz.transpose(0, 2, 1), points.transpose(0, 2, 1)], axis=-1)
    g1feat = _run_p0(xp, w0.T, bl)

    w0a = w0[:, :3].T
    b0v = b0.reshape(1, _D)
    z1, st1 = _run_p1(xyz, nx, g1feat, w0a, b0v, bl)
    st1 = lax.psum(st1, "c")

    gl1 = g0.reshape(1, _D)
    bel1 = be0.reshape(1, _D)
    gl2 = g1.reshape(1, _D)
    bel2 = be1.reshape(1, _D)
    gl3 = g2.reshape(1, 2 * _D)
    bel3 = be2.reshape(1, 2 * _D)
    b1v = b1.reshape(1, _D)
    b2v = b2.reshape(1, 2 * _D)
    w1t = w1.T
    w2t = w2.T

    st2 = _run_p2(z1, st1, w1t, b1v, gl1, bel1, bl)
    st2 = lax.psum(st2, "c")
    st3 = _run_p3(z1, st1, st2, w1t, b1v, gl1, bel1, gl2, bel2, w2t, b2v,
                  bl)
    st3 = lax.psum(st3, "c")
    pooled = _run_p4(z1, st1, st2, st3, w1t, b1v, gl1, bel1, gl2, bel2,
                     gl3, bel3, w2t, b2v, bl)

    new_xyz = nx[:, :, :3].transpose(0, 2, 1)
    new_points = pooled.transpose(0, 2, 1)
    return new_xyz, new_points


def kernel(xyz, points, w0, b0, g0, be0, w1, b1, g1, be1, w2, b2, g2, be2):
    devs = jax.devices()
    nd = 2 if (len(devs) >= 2 and _B % (2 * _G) == 0) else 1
    mesh = jax.sharding.Mesh(np.array(devs[:nd]), ("c",))
    shd = P("c")
    rep = P()
    f = _shard_map(
        _pipeline, mesh=mesh,
        in_specs=(shd, shd) + (rep,) * 12,
        out_specs=(shd, shd),
        check_rep=False)
    return f(xyz, points, w0, b0, g0, be0, w1, b1, g1, be1, w2, b2, g2,
             be2)


# speedup vs baseline: 1.1934x; 1.1934x over previous
import numpy as np

import jax
import jax.numpy as jnp
from jax import lax
from jax.experimental import pallas as pl
from jax.experimental.pallas import tpu as pltpu

try:
    from jax.experimental.shard_map import shard_map as _shard_map
except ImportError:
    _shard_map = jax.shard_map

P = jax.sharding.PartitionSpec

_NPOINT = 1024
_RADIUS = 0.2
_NSAMPLE = 32
_EPS = 1e-5
_B = 16
_N = 4096
_D = 64

_R2 = _RADIUS ** 2
_NROWS = _B * _NPOINT * _NSAMPLE

_F32 = jnp.float32


_G = 8
_SUB = 8
_LANE = _N // _SUB


_NCH = 4
_CW = _G // _NCH


def _fps_body(x_ref, y_ref, z_ref, *out_and_scratch):
    out_refs = out_and_scratch[:_G]
    scr_refs = out_and_scratch[_G:]
    nidx = (lax.broadcasted_iota(jnp.int32, (_CW, _SUB, _LANE), 1)
            * _LANE
            + lax.broadcasted_iota(jnp.int32, (_CW, _SUB, _LANE), 2)
            ).astype(_F32)
    li = lax.broadcasted_iota(jnp.int32, (_CW, 1, 128), 2)
    big = jnp.float32(1e10)
    xs = [x_ref[0, h * _CW:(h + 1) * _CW] for h in range(_NCH)]
    ys = [y_ref[0, h * _CW:(h + 1) * _CW] for h in range(_NCH)]
    zs = [z_ref[0, h * _CW:(h + 1) * _CW] for h in range(_NCH)]

    def step(i, carry):
        new_carry = []
        for h in range(_NCH):
            dists, onehot = carry[h]
            x, y, z = xs[h], ys[h], zs[h]
            px = jnp.sum(jnp.sum(x * onehot, axis=2, keepdims=True),
                         axis=1, keepdims=True)
            py = jnp.sum(jnp.sum(y * onehot, axis=2, keepdims=True),
                         axis=1, keepdims=True)
            pz = jnp.sum(jnp.sum(z * onehot, axis=2, keepdims=True),
                         axis=1, keepdims=True)
            rows = (jnp.where(li == 0, px, 0.0)
                    + jnp.where(li == 1, py, 0.0)
                    + jnp.where(li == 2, pz, 0.0))
            for g in range(_CW):
                scr_refs[h * _CW + g][pl.ds(i, 1), :, :] = rows[g:g + 1]
            d = ((x - px) ** 2 + (y - py) ** 2) + (z - pz) ** 2
            dists = jnp.minimum(dists, d)
            m = jnp.max(jnp.max(dists, axis=2, keepdims=True), axis=1,
                        keepdims=True)
            cand = jnp.where(dists == m, nidx, big)
            nxt = jnp.min(jnp.min(cand, axis=2, keepdims=True), axis=1,
                          keepdims=True)
            onehot = jnp.where(nidx == nxt, 1.0, 0.0)
            new_carry.append((dists, onehot))
        return tuple(new_carry)

    init1 = (jnp.full((_CW, _SUB, _LANE), big, _F32),
             jnp.where(nidx == 0.0, 1.0, 0.0))
    lax.fori_loop(0, _NPOINT, step, (init1,) * _NCH)
    for g in range(_G):
        out_refs[g][0] = scr_refs[g][...]


def _run_fps(xyz, bl):
    npg = bl // _G
    xr = xyz[:, 0, :].reshape(npg, _G, _SUB, _LANE)
    yr = xyz[:, 1, :].reshape(npg, _G, _SUB, _LANE)
    zr = xyz[:, 2, :].reshape(npg, _G, _SUB, _LANE)
    spec = pl.BlockSpec((1, _G, _SUB, _LANE), lambda c: (c, 0, 0, 0))
    out_spec = pl.BlockSpec((1, _NPOINT, 1, 128), lambda c: (c, 0, 0, 0))
    outs = pl.pallas_call(
        _fps_body,
        grid=(npg,),
        in_specs=[spec, spec, spec],
        out_specs=[out_spec] * _G,
        out_shape=[jax.ShapeDtypeStruct((npg, _NPOINT, 1, 128), _F32)] * _G,
        scratch_shapes=[pltpu.VMEM((_NPOINT, 1, 128), _F32)] * _G,
        compiler_params=pltpu.CompilerParams(
            dimension_semantics=("arbitrary",)),
    )(xr, yr, zr)
    nx = jnp.stack(outs, axis=1)[:, :, :, 0, :].reshape(bl, _NPOINT, 128)
    return nx


def _p0_body(xp_ref, w0t_ref, g1_ref):
    g1_ref[0] = jnp.dot(xp_ref[0], w0t_ref[...],
                        preferred_element_type=_F32)


def _run_p0(xp, w0t, bl):
    return pl.pallas_call(
        _p0_body,
        grid=(bl,),
        in_specs=[pl.BlockSpec((1, _N, 3 + _D), lambda b: (b, 0, 0)),
                  pl.BlockSpec((3 + _D, _D), lambda b: (0, 0))],
        out_specs=pl.BlockSpec((1, _N, _D), lambda b: (b, 0, 0)),
        out_shape=jax.ShapeDtypeStruct((bl, _N, _D), _F32),
        compiler_params=pltpu.CompilerParams(
            dimension_semantics=("arbitrary",)),
    )(xp, w0t)


_SBLK = 128
_NSB = _NPOINT // _SBLK
_SGRP = 8


def _p1_body(xyz_ref, nx_ref, g1_ref, w0a_ref, b0_ref, z1_ref, st_ref,
             sel_ref):
    x = xyz_ref[0, 0:1, :]
    y = xyz_ref[0, 1:2, :]
    z = xyz_ref[0, 2:3, :]
    xyz3 = xyz_ref[0]
    x2 = (x * x + y * y) + z * z
    wx = w0a_ref[0:1]
    wy = w0a_ref[1:2]
    wz = w0a_ref[2:3]
    b0 = b0_ref[...]
    kcol = lax.broadcasted_iota(
        jnp.int32, (_NSAMPLE, 1), 0).astype(_F32)
    kk3 = lax.broadcasted_iota(
        jnp.int32, (_SGRP, _NSAMPLE, 1), 1).astype(_F32)
    s1 = jnp.zeros((1, _D), _F32)
    s2 = jnp.zeros((1, _D), _F32)
    for i in range(_SBLK // _SGRP):
        c8 = nx_ref[0, i * _SGRP:(i + 1) * _SGRP, :]
        cx = c8[:, 0:1]
        cy = c8[:, 1:2]
        cz = c8[:, 2:3]
        mm = jnp.dot(c8[:, 0:3], xyz3, preferred_element_type=_F32)
        cs2 = (cx * cx + cy * cy) + cz * cz
        d = (-2.0 * mm + cs2) + x2
        maskf = jnp.where(d <= _R2, 1.0, 0.0)
        c = maskf
        sh = 1
        while sh < _N:
            c = c + jnp.concatenate(
                [jnp.zeros((_SGRP, sh), _F32), c[:, :-sh]], axis=1)
            sh *= 2
        cnt = c[:, _N - 1:_N]
        cm = c * maskf
        for s in range(_SGRP):
            sel_ref[s * _NSAMPLE:(s + 1) * _NSAMPLE, :] = jnp.where(
                cm[s:s + 1, :] == (kcol + 1.0), 1.0, 0.0)
        zres = jnp.dot(sel_ref[...], g1_ref[0],
                       preferred_element_type=_F32)
        z3 = zres.reshape(_SGRP, _NSAMPLE, _D)
        row0 = z3[:, 0:1, :]
        z3 = jnp.where(kk3 < cnt[:, :, None], z3, row0)
        t = (cx * wx + cy * wy + cz * wz) - b0
        z1f = (z3 - t[:, None, :]).reshape(_SGRP * _NSAMPLE, _D)
        z1_ref[0, i * 256:(i + 1) * 256, :] = z1f
        s1 = s1 + jnp.sum(z1f, axis=0, keepdims=True)
        s2 = s2 + jnp.sum(z1f * z1f, axis=0, keepdims=True)
    st_ref[0] = jnp.concatenate(
        [s1, s2, jnp.zeros((6, _D), _F32)], axis=0)


def _run_p1(xyz, nx, g1, w0a, b0v, bl):
    nblk = bl * _NSB
    return pl.pallas_call(
        _p1_body,
        grid=(bl, _NSB),
        in_specs=[
            pl.BlockSpec((1, 3, _N), lambda b, j: (b, 0, 0)),
            pl.BlockSpec((1, _SBLK, 128), lambda b, j: (b * _NSB + j, 0, 0)),
            pl.BlockSpec((1, _N, _D), lambda b, j: (b, 0, 0)),
            pl.BlockSpec((3, _D), lambda b, j: (0, 0)),
            pl.BlockSpec((1, _D), lambda b, j: (0, 0)),
        ],
        out_specs=[
            pl.BlockSpec((1, _SBLK * _NSAMPLE, _D),
                         lambda b, j: (b, j, 0)),
            pl.BlockSpec((1, 8, _D), lambda b, j: (b * _NSB + j, 0, 0)),
        ],
        out_shape=[
            jax.ShapeDtypeStruct((bl, _NPOINT * _NSAMPLE, _D), _F32),
            jax.ShapeDtypeStruct((nblk, 8, _D), _F32),
        ],
        scratch_shapes=[pltpu.VMEM((_SGRP * _NSAMPLE, _N), _F32)],
        compiler_params=pltpu.CompilerParams(
            dimension_semantics=("arbitrary", "arbitrary")),
    )(xyz, nx.reshape(bl * _NSB, _SBLK, 128), g1, w0a, b0v)


def _bn_affine(stv, gvec, bevec):
    s1 = jnp.sum(stv[:, 0, :], axis=0, keepdims=True)
    s2 = jnp.sum(stv[:, 1, :], axis=0, keepdims=True)
    mu = s1 / float(_NROWS)
    var = s2 / float(_NROWS) - mu * mu
    alpha = gvec * lax.rsqrt(var + _EPS)
    beta = bevec - mu * alpha
    return alpha, beta


_RBLK = 4096
_NRB = _NPOINT * _NSAMPLE // _RBLK


def _p2_body(z1_ref, st1_ref, w1t_ref, b1_ref, g1v_ref, be1_ref, st2_ref):
    al1, be1 = _bn_affine(st1_ref[...], g1v_ref[...], be1_ref[...])
    a = jnp.maximum(z1_ref[0] * al1 + be1, 0.0)
    z2 = jnp.dot(a, w1t_ref[...], preferred_element_type=_F32) + b1_ref[...]
    s1 = jnp.sum(z2, axis=0, keepdims=True)
    s2 = jnp.sum(z2 * z2, axis=0, keepdims=True)
    st2_ref[0] = jnp.concatenate(
        [s1, s2, jnp.zeros((6, _D), _F32)], axis=0)


def _run_p2(z1, st1, w1t, b1v, g1v, be1v, bl):
    nblk = bl * _NRB
    return pl.pallas_call(
        _p2_body,
        grid=(bl, _NRB),
        in_specs=[
            pl.BlockSpec((1, _RBLK, _D), lambda b, j: (b, j, 0)),
            pl.BlockSpec(st1.shape, lambda b, j: (0, 0, 0)),
            pl.BlockSpec((_D, _D), lambda b, j: (0, 0)),
            pl.BlockSpec((1, _D), lambda b, j: (0, 0)),
            pl.BlockSpec((1, _D), lambda b, j: (0, 0)),
            pl.BlockSpec((1, _D), lambda b, j: (0, 0)),
        ],
        out_specs=pl.BlockSpec((1, 8, _D), lambda b, j: (b * _NRB + j, 0, 0)),
        out_shape=jax.ShapeDtypeStruct((nblk, 8, _D), _F32),
        compiler_params=pltpu.CompilerParams(
            dimension_semantics=("arbitrary", "arbitrary")),
    )(z1, st1, w1t, b1v, g1v, be1v)


def _p3_body(z1_ref, st1_ref, st2_ref, w1t_ref, b1_ref, gl1_ref, bel1_ref,
             gl2_ref, bel2_ref, w2t_ref, b2_ref, st3_ref):
    al1, be1 = _bn_affine(st1_ref[...], gl1_ref[...], bel1_ref[...])
    a = jnp.maximum(z1_ref[0] * al1 + be1, 0.0)
    z2 = jnp.dot(a, w1t_ref[...], preferred_element_type=_F32) + b1_ref[...]
    al2, be2 = _bn_affine(st2_ref[...], gl2_ref[...], bel2_ref[...])
    a2 = jnp.maximum(z2 * al2 + be2, 0.0)
    z3 = jnp.dot(a2, w2t_ref[...], preferred_element_type=_F32) + b2_ref[...]
    s1 = jnp.sum(z3, axis=0, keepdims=True)
    s2 = jnp.sum(z3 * z3, axis=0, keepdims=True)
    st3_ref[0] = jnp.concatenate(
        [s1, s2, jnp.zeros((6, 2 * _D), _F32)], axis=0)


def _run_p3(z1, st1, st2, w1t, b1v, gl1, bel1, gl2, bel2, w2t, b2v, bl):
    nblk = bl * _NRB
    vec = lambda: pl.BlockSpec((1, _D), lambda b, j: (0, 0))
    return pl.pallas_call(
        _p3_body,
        grid=(bl, _NRB),
        in_specs=[
            pl.BlockSpec((1, _RBLK, _D), lambda b, j: (b, j, 0)),
            pl.BlockSpec(st1.shape, lambda b, j: (0, 0, 0)),
            pl.BlockSpec(st2.shape, lambda b, j: (0, 0, 0)),
            pl.BlockSpec((_D, _D), lambda b, j: (0, 0)),
            vec(), vec(), vec(), vec(), vec(),
            pl.BlockSpec((_D, 2 * _D), lambda b, j: (0, 0)),
            pl.BlockSpec((1, 2 * _D), lambda b, j: (0, 0)),
        ],
        out_specs=pl.BlockSpec((1, 8, 2 * _D),
                               lambda b, j: (b * _NRB + j, 0, 0)),
        out_shape=jax.ShapeDtypeStruct((nblk, 8, 2 * _D), _F32),
        compiler_params=pltpu.CompilerParams(
            dimension_semantics=("arbitrary", "arbitrary")),
    )(z1, st1, st2, w1t, b1v, gl1, bel1, gl2, bel2, w2t, b2v)


def _p4_body(z1_ref, st1_ref, st2_ref, st3_ref, w1t_ref, b1_ref,
             gl1_ref, bel1_ref, gl2_ref, bel2_ref, gl3_ref, bel3_ref,
             w2t_ref, b2_ref, out_ref):
    al1, be1 = _bn_affine(st1_ref[...], gl1_ref[...], bel1_ref[...])
    a = jnp.maximum(z1_ref[0] * al1 + be1, 0.0)
    z2 = jnp.dot(a, w1t_ref[...], preferred_element_type=_F32) + b1_ref[...]
    al2, be2 = _bn_affine(st2_ref[...], gl2_ref[...], bel2_ref[...])
    a2 = jnp.maximum(z2 * al2 + be2, 0.0)
    z3 = jnp.dot(a2, w2t_ref[...], preferred_element_type=_F32) + b2_ref[...]
    al3, be3 = _bn_affine(st3_ref[...], gl3_ref[...], bel3_ref[...])
    a3 = jnp.maximum(z3 * al3 + be3, 0.0)
    pooled = jnp.max(a3.reshape(_RBLK // _NSAMPLE, _NSAMPLE, 2 * _D),
                     axis=1)
    out_ref[0] = pooled


def _run_p4(z1, st1, st2, st3, w1t, b1v, gl1, bel1, gl2, bel2, gl3, bel3,
            w2t, b2v, bl):
    vec = lambda: pl.BlockSpec((1, _D), lambda b, j: (0, 0))
    vec2 = lambda: pl.BlockSpec((1, 2 * _D), lambda b, j: (0, 0))
    return pl.pallas_call(
        _p4_body,
        grid=(bl, _NRB),
        in_specs=[
            pl.BlockSpec((1, _RBLK, _D), lambda b, j: (b, j, 0)),
            pl.BlockSpec(st1.shape, lambda b, j: (0, 0, 0)),
            pl.BlockSpec(st2.shape, lambda b, j: (0, 0, 0)),
            pl.BlockSpec(st3.shape, lambda b, j: (0, 0, 0)),
            pl.BlockSpec((_D, _D), lambda b, j: (0, 0)),
            vec(), vec(), vec(), vec(), vec(),
            vec2(), vec2(),
            pl.BlockSpec((_D, 2 * _D), lambda b, j: (0, 0)),
            vec2(),
        ],
        out_specs=pl.BlockSpec((1, _RBLK // _NSAMPLE, 2 * _D),
                               lambda b, j: (b, j, 0)),
        out_shape=jax.ShapeDtypeStruct((bl, _NPOINT, 2 * _D), _F32),
        compiler_params=pltpu.CompilerParams(
            dimension_semantics=("arbitrary", "arbitrary")),
    )(z1, st1, st2, st3, w1t, b1v, gl1, bel1, gl2, bel2, gl3, bel3,
      w2t, b2v)


def _pipeline(xyz, points, w0, b0, g0, be0, w1, b1, g1, be1, w2, b2, g2,
              be2):
    bl = xyz.shape[0]
    xyz = xyz.astype(_F32)
    points = points.astype(_F32)

    nx = _run_fps(xyz, bl)

    xp = jnp.concatenate(
        [xyz.transpose(0, 2, 1), points.transpose(0, 2, 1)], axis=-1)
    g1feat = _run_p0(xp, w0.T, bl)

    w0a = w0[:, :3].T
    b0v = b0.reshape(1, _D)
    z1, st1 = _run_p1(xyz, nx, g1feat, w0a, b0v, bl)
    st1 = lax.psum(st1, "c")

    gl1 = g0.reshape(1, _D)
    bel1 = be0.reshape(1, _D)
    gl2 = g1.reshape(1, _D)
    bel2 = be1.reshape(1, _D)
    gl3 = g2.reshape(1, 2 * _D)
    bel3 = be2.reshape(1, 2 * _D)
    b1v = b1.reshape(1, _D)
    b2v = b2.reshape(1, 2 * _D)
    w1t = w1.T
    w2t = w2.T

    st2 = _run_p2(z1, st1, w1t, b1v, gl1, bel1, bl)
    st2 = lax.psum(st2, "c")
    st3 = _run_p3(z1, st1, st2, w1t, b1v, gl1, bel1, gl2, bel2, w2t, b2v,
                  bl)
    st3 = lax.psum(st3, "c")
    pooled = _run_p4(z1, st1, st2, st3, w1t, b1v, gl1, bel1, gl2, bel2,
                     gl3, bel3, w2t, b2v, bl)

    new_xyz = nx[:, :, :3].transpose(0, 2, 1)
    new_points = pooled.transpose(0, 2, 1)
    return new_xyz, new_points


def kernel(xyz, points, w0, b0, g0, be0, w1, b1, g1, be1, w2, b2, g2, be2):
    devs = jax.devices()
    nd = 2 if (len(devs) >= 2 and _B % (2 * _G) == 0) else 1
    mesh = jax.sharding.Mesh(np.array(devs[:nd]), ("c",))
    shd = P("c")
    rep = P()
    f = _shard_map(
        _pipeline, mesh=mesh,
        in_specs=(shd, shd) + (rep,) * 12,
        out_specs=(shd, shd),
        check_rep=False)
    return f(xyz, points, w0, b0, g0, be0, w1, b1, g1, be1, w2, b2, g2,
             be2)


# speedup vs baseline: 1.2311x; 1.0316x over previous
import numpy as np

import jax
import jax.numpy as jnp
from jax import lax
from jax.experimental import pallas as pl
from jax.experimental.pallas import tpu as pltpu

try:
    from jax.experimental.shard_map import shard_map as _shard_map
except ImportError:
    _shard_map = jax.shard_map

P = jax.sharding.PartitionSpec

_NPOINT = 1024
_RADIUS = 0.2
_NSAMPLE = 32
_EPS = 1e-5
_B = 16
_N = 4096
_D = 64

_R2 = _RADIUS ** 2
_NROWS = _B * _NPOINT * _NSAMPLE

_F32 = jnp.float32


_G = 8
_SUB = 8
_LANE = _N // _SUB


_NCH = 4
_CW = _G // _NCH


def _fps_body(x_ref, y_ref, z_ref, *out_and_scratch):
    out_refs = out_and_scratch[:_G]
    scr_refs = out_and_scratch[_G:]
    nidx = (lax.broadcasted_iota(jnp.int32, (_CW, _SUB, _LANE), 1)
            * _LANE
            + lax.broadcasted_iota(jnp.int32, (_CW, _SUB, _LANE), 2)
            ).astype(_F32)
    li = lax.broadcasted_iota(jnp.int32, (_CW, 1, 128), 2)
    big = jnp.float32(1e10)
    xs = [x_ref[0, h * _CW:(h + 1) * _CW] for h in range(_NCH)]
    ys = [y_ref[0, h * _CW:(h + 1) * _CW] for h in range(_NCH)]
    zs = [z_ref[0, h * _CW:(h + 1) * _CW] for h in range(_NCH)]

    def step(i, carry):
        new_carry = []
        for h in range(_NCH):
            dists, onehot = carry[h]
            x, y, z = xs[h], ys[h], zs[h]
            px = jnp.sum(jnp.sum(x * onehot, axis=2, keepdims=True),
                         axis=1, keepdims=True)
            py = jnp.sum(jnp.sum(y * onehot, axis=2, keepdims=True),
                         axis=1, keepdims=True)
            pz = jnp.sum(jnp.sum(z * onehot, axis=2, keepdims=True),
                         axis=1, keepdims=True)
            rows = (jnp.where(li == 0, px, 0.0)
                    + jnp.where(li == 1, py, 0.0)
                    + jnp.where(li == 2, pz, 0.0))
            for g in range(_CW):
                scr_refs[h * _CW + g][pl.ds(i, 1), :, :] = rows[g:g + 1]
            d = ((x - px) ** 2 + (y - py) ** 2) + (z - pz) ** 2
            dists = jnp.minimum(dists, d)
            m = jnp.max(jnp.max(dists, axis=2, keepdims=True), axis=1,
                        keepdims=True)
            cand = jnp.where(dists == m, nidx, big)
            nxt = jnp.min(jnp.min(cand, axis=2, keepdims=True), axis=1,
                          keepdims=True)
            onehot = jnp.where(nidx == nxt, 1.0, 0.0)
            new_carry.append((dists, onehot))
        return tuple(new_carry)

    init1 = (jnp.full((_CW, _SUB, _LANE), big, _F32),
             jnp.where(nidx == 0.0, 1.0, 0.0))
    lax.fori_loop(0, _NPOINT, step, (init1,) * _NCH)
    for g in range(_G):
        out_refs[g][0] = scr_refs[g][...]


def _run_fps(xyz, bl):
    npg = bl // _G
    xr = xyz[:, 0, :].reshape(npg, _G, _SUB, _LANE)
    yr = xyz[:, 1, :].reshape(npg, _G, _SUB, _LANE)
    zr = xyz[:, 2, :].reshape(npg, _G, _SUB, _LANE)
    spec = pl.BlockSpec((1, _G, _SUB, _LANE), lambda c: (c, 0, 0, 0))
    out_spec = pl.BlockSpec((1, _NPOINT, 1, 128), lambda c: (c, 0, 0, 0))
    outs = pl.pallas_call(
        _fps_body,
        grid=(npg,),
        in_specs=[spec, spec, spec],
        out_specs=[out_spec] * _G,
        out_shape=[jax.ShapeDtypeStruct((npg, _NPOINT, 1, 128), _F32)] * _G,
        scratch_shapes=[pltpu.VMEM((_NPOINT, 1, 128), _F32)] * _G,
        compiler_params=pltpu.CompilerParams(
            dimension_semantics=("arbitrary",)),
    )(xr, yr, zr)
    nx = jnp.stack(outs, axis=1)[:, :, :, 0, :].reshape(bl, _NPOINT, 128)
    return nx


def _p0_body(xyz_ref, pts_ref, w0a_ref, w0b_ref, g1_ref):
    dn = (((0,), (0,)), ((), ()))
    g1_ref[0] = (lax.dot_general(xyz_ref[0], w0a_ref[...], dn,
                                 preferred_element_type=_F32)
                 + lax.dot_general(pts_ref[0], w0b_ref[...], dn,
                                   preferred_element_type=_F32))


def _run_p0(xyz, points, w0a, w0b, bl):
    return pl.pallas_call(
        _p0_body,
        grid=(bl,),
        in_specs=[pl.BlockSpec((1, 3, _N), lambda b: (b, 0, 0)),
                  pl.BlockSpec((1, _D, _N), lambda b: (b, 0, 0)),
                  pl.BlockSpec((3, _D), lambda b: (0, 0)),
                  pl.BlockSpec((_D, _D), lambda b: (0, 0))],
        out_specs=pl.BlockSpec((1, _N, _D), lambda b: (b, 0, 0)),
        out_shape=jax.ShapeDtypeStruct((bl, _N, _D), _F32),
        compiler_params=pltpu.CompilerParams(
            dimension_semantics=("arbitrary",)),
    )(xyz, points, w0a, w0b)


_SBLK = 128
_NSB = _NPOINT // _SBLK
_SGRP = 8


def _p1_body(xyz_ref, nx_ref, g1_ref, w0a_ref, b0_ref, z1_ref, st_ref,
             sel_ref_a, sel_ref_b):
    x = xyz_ref[0, 0:1, :]
    y = xyz_ref[0, 1:2, :]
    z = xyz_ref[0, 2:3, :]
    xyz3 = xyz_ref[0]
    x2 = (x * x + y * y) + z * z
    wx = w0a_ref[0:1]
    wy = w0a_ref[1:2]
    wz = w0a_ref[2:3]
    b0 = b0_ref[...]
    kcol = lax.broadcasted_iota(
        jnp.int32, (_NSAMPLE, 1), 0).astype(_F32)
    kk3 = lax.broadcasted_iota(
        jnp.int32, (_SGRP, _NSAMPLE, 1), 1).astype(_F32)
    s1 = jnp.zeros((1, _D), _F32)
    s2 = jnp.zeros((1, _D), _F32)
    for i in range(_SBLK // _SGRP):
        c8 = nx_ref[0, i * _SGRP:(i + 1) * _SGRP, :]
        cx = c8[:, 0:1]
        cy = c8[:, 1:2]
        cz = c8[:, 2:3]
        mm = jnp.dot(c8[:, 0:3], xyz3, preferred_element_type=_F32)
        cs2 = (cx * cx + cy * cy) + cz * cz
        d = (-2.0 * mm + cs2) + x2
        maskf = jnp.where(d <= _R2, 1.0, 0.0)
        c = maskf
        sh = 1
        while sh < _N:
            c = c + jnp.concatenate(
                [jnp.zeros((_SGRP, sh), _F32), c[:, :-sh]], axis=1)
            sh *= 2
        cnt = c[:, _N - 1:_N]
        cm = c * maskf
        sel_ref = sel_ref_a if i % 2 == 0 else sel_ref_b
        for s in range(_SGRP):
            sel_ref[s * _NSAMPLE:(s + 1) * _NSAMPLE, :] = jnp.where(
                cm[s:s + 1, :] == (kcol + 1.0), 1.0, 0.0)
        zres = jnp.dot(sel_ref[...], g1_ref[0],
                       preferred_element_type=_F32)
        z3 = zres.reshape(_SGRP, _NSAMPLE, _D)
        row0 = z3[:, 0:1, :]
        z3 = jnp.where(kk3 < cnt[:, :, None], z3, row0)
        t = (cx * wx + cy * wy + cz * wz) - b0
        z1f = (z3 - t[:, None, :]).reshape(_SGRP * _NSAMPLE, _D)
        z1_ref[0, i * 256:(i + 1) * 256, :] = z1f
        s1 = s1 + jnp.sum(z1f, axis=0, keepdims=True)
        s2 = s2 + jnp.sum(z1f * z1f, axis=0, keepdims=True)
    st_ref[0] = jnp.concatenate(
        [s1, s2, jnp.zeros((6, _D), _F32)], axis=0)


def _run_p1(xyz, nx, g1, w0a, b0v, bl):
    nblk = bl * _NSB
    return pl.pallas_call(
        _p1_body,
        grid=(bl, _NSB),
        in_specs=[
            pl.BlockSpec((1, 3, _N), lambda b, j: (b, 0, 0)),
            pl.BlockSpec((1, _SBLK, 128), lambda b, j: (b * _NSB + j, 0, 0)),
            pl.BlockSpec((1, _N, _D), lambda b, j: (b, 0, 0)),
            pl.BlockSpec((3, _D), lambda b, j: (0, 0)),
            pl.BlockSpec((1, _D), lambda b, j: (0, 0)),
        ],
        out_specs=[
            pl.BlockSpec((1, _SBLK * _NSAMPLE, _D),
                         lambda b, j: (b, j, 0)),
            pl.BlockSpec((1, 8, _D), lambda b, j: (b * _NSB + j, 0, 0)),
        ],
        out_shape=[
            jax.ShapeDtypeStruct((bl, _NPOINT * _NSAMPLE, _D), _F32),
            jax.ShapeDtypeStruct((nblk, 8, _D), _F32),
        ],
        scratch_shapes=[pltpu.VMEM((_SGRP * _NSAMPLE, _N), _F32)] * 2,
        compiler_params=pltpu.CompilerParams(
            dimension_semantics=("arbitrary", "arbitrary")),
    )(xyz, nx.reshape(bl * _NSB, _SBLK, 128), g1, w0a, b0v)


def _bn_affine(stv, gvec, bevec):
    s1 = jnp.sum(stv[:, 0, :], axis=0, keepdims=True)
    s2 = jnp.sum(stv[:, 1, :], axis=0, keepdims=True)
    mu = s1 / float(_NROWS)
    var = s2 / float(_NROWS) - mu * mu
    alpha = gvec * lax.rsqrt(var + _EPS)
    beta = bevec - mu * alpha
    return alpha, beta


_RBLK = 4096
_NRB = _NPOINT * _NSAMPLE // _RBLK


def _p2_body(z1_ref, st1_ref, w1t_ref, b1_ref, g1v_ref, be1_ref, st2_ref):
    al1, be1 = _bn_affine(st1_ref[...], g1v_ref[...], be1_ref[...])
    a = jnp.maximum(z1_ref[0] * al1 + be1, 0.0)
    z2 = jnp.dot(a, w1t_ref[...], preferred_element_type=_F32) + b1_ref[...]
    s1 = jnp.sum(z2, axis=0, keepdims=True)
    s2 = jnp.sum(z2 * z2, axis=0, keepdims=True)
    st2_ref[0] = jnp.concatenate(
        [s1, s2, jnp.zeros((6, _D), _F32)], axis=0)


def _run_p2(z1, st1, w1t, b1v, g1v, be1v, bl):
    nblk = bl * _NRB
    return pl.pallas_call(
        _p2_body,
        grid=(bl, _NRB),
        in_specs=[
            pl.BlockSpec((1, _RBLK, _D), lambda b, j: (b, j, 0)),
            pl.BlockSpec(st1.shape, lambda b, j: (0, 0, 0)),
            pl.BlockSpec((_D, _D), lambda b, j: (0, 0)),
            pl.BlockSpec((1, _D), lambda b, j: (0, 0)),
            pl.BlockSpec((1, _D), lambda b, j: (0, 0)),
            pl.BlockSpec((1, _D), lambda b, j: (0, 0)),
        ],
        out_specs=pl.BlockSpec((1, 8, _D), lambda b, j: (b * _NRB + j, 0, 0)),
        out_shape=jax.ShapeDtypeStruct((nblk, 8, _D), _F32),
        compiler_params=pltpu.CompilerParams(
            dimension_semantics=("arbitrary", "arbitrary")),
    )(z1, st1, w1t, b1v, g1v, be1v)


def _p3_body(z1_ref, st1_ref, st2_ref, w1t_ref, b1_ref, gl1_ref, bel1_ref,
             gl2_ref, bel2_ref, w2t_ref, b2_ref, st3_ref):
    al1, be1 = _bn_affine(st1_ref[...], gl1_ref[...], bel1_ref[...])
    a = jnp.maximum(z1_ref[0] * al1 + be1, 0.0)
    z2 = jnp.dot(a, w1t_ref[...], preferred_element_type=_F32) + b1_ref[...]
    al2, be2 = _bn_affine(st2_ref[...], gl2_ref[...], bel2_ref[...])
    a2 = jnp.maximum(z2 * al2 + be2, 0.0)
    z3 = jnp.dot(a2, w2t_ref[...], preferred_element_type=_F32) + b2_ref[...]
    s1 = jnp.sum(z3, axis=0, keepdims=True)
    s2 = jnp.sum(z3 * z3, axis=0, keepdims=True)
    st3_ref[0] = jnp.concatenate(
        [s1, s2, jnp.zeros((6, 2 * _D), _F32)], axis=0)


def _run_p3(z1, st1, st2, w1t, b1v, gl1, bel1, gl2, bel2, w2t, b2v, bl):
    nblk = bl * _NRB
    vec = lambda: pl.BlockSpec((1, _D), lambda b, j: (0, 0))
    return pl.pallas_call(
        _p3_body,
        grid=(bl, _NRB),
        in_specs=[
            pl.BlockSpec((1, _RBLK, _D), lambda b, j: (b, j, 0)),
            pl.BlockSpec(st1.shape, lambda b, j: (0, 0, 0)),
            pl.BlockSpec(st2.shape, lambda b, j: (0, 0, 0)),
            pl.BlockSpec((_D, _D), lambda b, j: (0, 0)),
            vec(), vec(), vec(), vec(), vec(),
            pl.BlockSpec((_D, 2 * _D), lambda b, j: (0, 0)),
            pl.BlockSpec((1, 2 * _D), lambda b, j: (0, 0)),
        ],
        out_specs=pl.BlockSpec((1, 8, 2 * _D),
                               lambda b, j: (b * _NRB + j, 0, 0)),
        out_shape=jax.ShapeDtypeStruct((nblk, 8, 2 * _D), _F32),
        compiler_params=pltpu.CompilerParams(
            dimension_semantics=("arbitrary", "arbitrary")),
    )(z1, st1, st2, w1t, b1v, gl1, bel1, gl2, bel2, w2t, b2v)


def _p4_body(z1_ref, st1_ref, st2_ref, st3_ref, w1t_ref, b1_ref,
             gl1_ref, bel1_ref, gl2_ref, bel2_ref, gl3_ref, bel3_ref,
             w2t_ref, b2_ref, out_ref):
    al1, be1 = _bn_affine(st1_ref[...], gl1_ref[...], bel1_ref[...])
    a = jnp.maximum(z1_ref[0] * al1 + be1, 0.0)
    z2 = jnp.dot(a, w1t_ref[...], preferred_element_type=_F32) + b1_ref[...]
    al2, be2 = _bn_affine(st2_ref[...], gl2_ref[...], bel2_ref[...])
    a2 = jnp.maximum(z2 * al2 + be2, 0.0)
    z3 = jnp.dot(a2, w2t_ref[...], preferred_element_type=_F32) + b2_ref[...]
    al3, be3 = _bn_affine(st3_ref[...], gl3_ref[...], bel3_ref[...])
    a3 = jnp.maximum(z3 * al3 + be3, 0.0)
    pooled = jnp.max(a3.reshape(_RBLK // _NSAMPLE, _NSAMPLE, 2 * _D),
                     axis=1)
    out_ref[0] = pooled


def _run_p4(z1, st1, st2, st3, w1t, b1v, gl1, bel1, gl2, bel2, gl3, bel3,
            w2t, b2v, bl):
    vec = lambda: pl.BlockSpec((1, _D), lambda b, j: (0, 0))
    vec2 = lambda: pl.BlockSpec((1, 2 * _D), lambda b, j: (0, 0))
    return pl.pallas_call(
        _p4_body,
        grid=(bl, _NRB),
        in_specs=[
            pl.BlockSpec((1, _RBLK, _D), lambda b, j: (b, j, 0)),
            pl.BlockSpec(st1.shape, lambda b, j: (0, 0, 0)),
            pl.BlockSpec(st2.shape, lambda b, j: (0, 0, 0)),
            pl.BlockSpec(st3.shape, lambda b, j: (0, 0, 0)),
            pl.BlockSpec((_D, _D), lambda b, j: (0, 0)),
            vec(), vec(), vec(), vec(), vec(),
            vec2(), vec2(),
            pl.BlockSpec((_D, 2 * _D), lambda b, j: (0, 0)),
            vec2(),
        ],
        out_specs=pl.BlockSpec((1, _RBLK // _NSAMPLE, 2 * _D),
                               lambda b, j: (b, j, 0)),
        out_shape=jax.ShapeDtypeStruct((bl, _NPOINT, 2 * _D), _F32),
        compiler_params=pltpu.CompilerParams(
            dimension_semantics=("arbitrary", "arbitrary")),
    )(z1, st1, st2, st3, w1t, b1v, gl1, bel1, gl2, bel2, gl3, bel3,
      w2t, b2v)


def _pipeline(xyz, points, w0, b0, g0, be0, w1, b1, g1, be1, w2, b2, g2,
              be2):
    bl = xyz.shape[0]
    xyz = xyz.astype(_F32)
    points = points.astype(_F32)

    nx = _run_fps(xyz, bl)

    w0a = w0[:, :3].T
    g1feat = _run_p0(xyz, points, w0a, w0[:, 3:].T, bl)

    b0v = b0.reshape(1, _D)
    z1, st1 = _run_p1(xyz, nx, g1feat, w0a, b0v, bl)
    st1 = lax.psum(st1, "c")

    gl1 = g0.reshape(1, _D)
    bel1 = be0.reshape(1, _D)
    gl2 = g1.reshape(1, _D)
    bel2 = be1.reshape(1, _D)
    gl3 = g2.reshape(1, 2 * _D)
    bel3 = be2.reshape(1, 2 * _D)
    b1v = b1.reshape(1, _D)
    b2v = b2.reshape(1, 2 * _D)
    w1t = w1.T
    w2t = w2.T

    st2 = _run_p2(z1, st1, w1t, b1v, gl1, bel1, bl)
    st2 = lax.psum(st2, "c")
    st3 = _run_p3(z1, st1, st2, w1t, b1v, gl1, bel1, gl2, bel2, w2t, b2v,
                  bl)
    st3 = lax.psum(st3, "c")
    pooled = _run_p4(z1, st1, st2, st3, w1t, b1v, gl1, bel1, gl2, bel2,
                     gl3, bel3, w2t, b2v, bl)

    new_xyz = nx[:, :, :3].transpose(0, 2, 1)
    new_points = pooled.transpose(0, 2, 1)
    return new_xyz, new_points


def kernel(xyz, points, w0, b0, g0, be0, w1, b1, g1, be1, w2, b2, g2, be2):
    devs = jax.devices()
    nd = 2 if (len(devs) >= 2 and _B % (2 * _G) == 0) else 1
    mesh = jax.sharding.Mesh(np.array(devs[:nd]), ("c",))
    shd = P("c")
    rep = P()
    f = _shard_map(
        _pipeline, mesh=mesh,
        in_specs=(shd, shd) + (rep,) * 12,
        out_specs=(shd, shd),
        check_rep=False)
    return f(xyz, points, w0, b0, g0, be0, w1, b1, g1, be1, w2, b2, g2,
             be2)


# speedup vs baseline: 1.2514x; 1.0165x over previous
import numpy as np

import jax
import jax.numpy as jnp
from jax import lax
from jax.experimental import pallas as pl
from jax.experimental.pallas import tpu as pltpu

try:
    from jax.experimental.shard_map import shard_map as _shard_map
except ImportError:
    _shard_map = jax.shard_map

P = jax.sharding.PartitionSpec

_NPOINT = 1024
_RADIUS = 0.2
_NSAMPLE = 32
_EPS = 1e-5
_B = 16
_N = 4096
_D = 64

_R2 = _RADIUS ** 2
_NROWS = _B * _NPOINT * _NSAMPLE

_F32 = jnp.float32


_G = 8
_SUB = 8
_LANE = _N // _SUB


_NCH = 4
_CW = _G // _NCH


def _fps_body(x_ref, y_ref, z_ref, *out_and_scratch):
    out_refs = out_and_scratch[:_G]
    scr_refs = out_and_scratch[_G:]
    nidx = (lax.broadcasted_iota(jnp.int32, (_CW, _SUB, _LANE), 1)
            * _LANE
            + lax.broadcasted_iota(jnp.int32, (_CW, _SUB, _LANE), 2)
            ).astype(_F32)
    li = lax.broadcasted_iota(jnp.int32, (_CW, 1, 128), 2)
    big = jnp.float32(1e10)
    xs = [x_ref[0, h * _CW:(h + 1) * _CW] for h in range(_NCH)]
    ys = [y_ref[0, h * _CW:(h + 1) * _CW] for h in range(_NCH)]
    zs = [z_ref[0, h * _CW:(h + 1) * _CW] for h in range(_NCH)]

    def step(i, carry):
        new_carry = []
        for h in range(_NCH):
            dists, onehot = carry[h]
            x, y, z = xs[h], ys[h], zs[h]
            px = jnp.sum(jnp.sum(x * onehot, axis=2, keepdims=True),
                         axis=1, keepdims=True)
            py = jnp.sum(jnp.sum(y * onehot, axis=2, keepdims=True),
                         axis=1, keepdims=True)
            pz = jnp.sum(jnp.sum(z * onehot, axis=2, keepdims=True),
                         axis=1, keepdims=True)
            rows = (jnp.where(li == 0, px, 0.0)
                    + jnp.where(li == 1, py, 0.0)
                    + jnp.where(li == 2, pz, 0.0))
            for g in range(_CW):
                scr_refs[h * _CW + g][pl.ds(i, 1), :, :] = rows[g:g + 1]
            d = ((x - px) ** 2 + (y - py) ** 2) + (z - pz) ** 2
            dists = jnp.minimum(dists, d)
            m = jnp.max(jnp.max(dists, axis=2, keepdims=True), axis=1,
                        keepdims=True)
            cand = jnp.where(dists == m, nidx, big)
            nxt = jnp.min(jnp.min(cand, axis=2, keepdims=True), axis=1,
                          keepdims=True)
            onehot = jnp.where(nidx == nxt, 1.0, 0.0)
            new_carry.append((dists, onehot))
        return tuple(new_carry)

    init1 = (jnp.full((_CW, _SUB, _LANE), big, _F32),
             jnp.where(nidx == 0.0, 1.0, 0.0))
    lax.fori_loop(0, _NPOINT, step, (init1,) * _NCH)
    for g in range(_G):
        out_refs[g][0] = scr_refs[g][...]


def _run_fps(xyz, bl):
    npg = bl // _G
    xr = xyz[:, 0, :].reshape(npg, _G, _SUB, _LANE)
    yr = xyz[:, 1, :].reshape(npg, _G, _SUB, _LANE)
    zr = xyz[:, 2, :].reshape(npg, _G, _SUB, _LANE)
    spec = pl.BlockSpec((1, _G, _SUB, _LANE), lambda c: (c, 0, 0, 0))
    out_spec = pl.BlockSpec((1, _NPOINT, 1, 128), lambda c: (c, 0, 0, 0))
    outs = pl.pallas_call(
        _fps_body,
        grid=(npg,),
        in_specs=[spec, spec, spec],
        out_specs=[out_spec] * _G,
        out_shape=[jax.ShapeDtypeStruct((npg, _NPOINT, 1, 128), _F32)] * _G,
        scratch_shapes=[pltpu.VMEM((_NPOINT, 1, 128), _F32)] * _G,
        compiler_params=pltpu.CompilerParams(
            dimension_semantics=("arbitrary",)),
    )(xr, yr, zr)
    nx = jnp.stack(outs, axis=1)[:, :, :, 0, :].reshape(bl, _NPOINT, 128)
    return nx


def _p0_body(xyz_ref, pts_ref, w0a_ref, w0b_ref, g1_ref):
    dn = (((0,), (0,)), ((), ()))
    g1_ref[0] = (lax.dot_general(xyz_ref[0], w0a_ref[...], dn,
                                 preferred_element_type=_F32)
                 + lax.dot_general(pts_ref[0], w0b_ref[...], dn,
                                   preferred_element_type=_F32))


def _run_p0(xyz, points, w0a, w0b, bl):
    return pl.pallas_call(
        _p0_body,
        grid=(bl,),
        in_specs=[pl.BlockSpec((1, 3, _N), lambda b: (b, 0, 0)),
                  pl.BlockSpec((1, _D, _N), lambda b: (b, 0, 0)),
                  pl.BlockSpec((3, _D), lambda b: (0, 0)),
                  pl.BlockSpec((_D, _D), lambda b: (0, 0))],
        out_specs=pl.BlockSpec((1, _N, _D), lambda b: (b, 0, 0)),
        out_shape=jax.ShapeDtypeStruct((bl, _N, _D), _F32),
        compiler_params=pltpu.CompilerParams(
            dimension_semantics=("arbitrary",)),
    )(xyz, points, w0a, w0b)


_SBLK = 128
_NSB = _NPOINT // _SBLK
_SGRP = 16


def _p1_body(xyz_ref, nx_ref, g1_ref, w0a_ref, b0_ref, z1_ref, st_ref,
             sel_ref_a, sel_ref_b):
    x = xyz_ref[0, 0:1, :]
    y = xyz_ref[0, 1:2, :]
    z = xyz_ref[0, 2:3, :]
    xyz3 = xyz_ref[0]
    x2 = (x * x + y * y) + z * z
    wx = w0a_ref[0:1]
    wy = w0a_ref[1:2]
    wz = w0a_ref[2:3]
    b0 = b0_ref[...]
    kcol = lax.broadcasted_iota(
        jnp.int32, (_NSAMPLE, 1), 0).astype(_F32)
    kk3 = lax.broadcasted_iota(
        jnp.int32, (_SGRP, _NSAMPLE, 1), 1).astype(_F32)
    s1 = jnp.zeros((1, _D), _F32)
    s2 = jnp.zeros((1, _D), _F32)
    for i in range(_SBLK // _SGRP):
        c8 = nx_ref[0, i * _SGRP:(i + 1) * _SGRP, :]
        cx = c8[:, 0:1]
        cy = c8[:, 1:2]
        cz = c8[:, 2:3]
        mm = jnp.dot(c8[:, 0:3], xyz3, preferred_element_type=_F32)
        cs2 = (cx * cx + cy * cy) + cz * cz
        d = (-2.0 * mm + cs2) + x2
        maskf = jnp.where(d <= _R2, 1.0, 0.0)
        c = maskf
        sh = 1
        while sh < _N:
            c = c + jnp.concatenate(
                [jnp.zeros((_SGRP, sh), _F32), c[:, :-sh]], axis=1)
            sh *= 2
        cnt = c[:, _N - 1:_N]
        cm = c * maskf
        sel_ref = sel_ref_a if i % 2 == 0 else sel_ref_b
        for s in range(_SGRP):
            sel_ref[s * _NSAMPLE:(s + 1) * _NSAMPLE, :] = jnp.where(
                cm[s:s + 1, :] == (kcol + 1.0), 1.0, 0.0)
        zres = jnp.dot(sel_ref[...], g1_ref[0],
                       preferred_element_type=_F32)
        z3 = zres.reshape(_SGRP, _NSAMPLE, _D)
        row0 = z3[:, 0:1, :]
        z3 = jnp.where(kk3 < cnt[:, :, None], z3, row0)
        t = (cx * wx + cy * wy + cz * wz) - b0
        z1f = (z3 - t[:, None, :]).reshape(_SGRP * _NSAMPLE, _D)
        rpg = _SGRP * _NSAMPLE
        z1_ref[0, i * rpg:(i + 1) * rpg, :] = z1f
        s1 = s1 + jnp.sum(z1f, axis=0, keepdims=True)
        s2 = s2 + jnp.sum(z1f * z1f, axis=0, keepdims=True)
    st_ref[0] = jnp.concatenate(
        [s1, s2, jnp.zeros((6, _D), _F32)], axis=0)


def _run_p1(xyz, nx, g1, w0a, b0v, bl):
    nblk = bl * _NSB
    return pl.pallas_call(
        _p1_body,
        grid=(bl, _NSB),
        in_specs=[
            pl.BlockSpec((1, 3, _N), lambda b, j: (b, 0, 0)),
            pl.BlockSpec((1, _SBLK, 128), lambda b, j: (b * _NSB + j, 0, 0)),
            pl.BlockSpec((1, _N, _D), lambda b, j: (b, 0, 0)),
            pl.BlockSpec((3, _D), lambda b, j: (0, 0)),
            pl.BlockSpec((1, _D), lambda b, j: (0, 0)),
        ],
        out_specs=[
            pl.BlockSpec((1, _SBLK * _NSAMPLE, _D),
                         lambda b, j: (b, j, 0)),
            pl.BlockSpec((1, 8, _D), lambda b, j: (b * _NSB + j, 0, 0)),
        ],
        out_shape=[
            jax.ShapeDtypeStruct((bl, _NPOINT * _NSAMPLE, _D), _F32),
            jax.ShapeDtypeStruct((nblk, 8, _D), _F32),
        ],
        scratch_shapes=[pltpu.VMEM((_SGRP * _NSAMPLE, _N), _F32)] * 2,
        compiler_params=pltpu.CompilerParams(
            dimension_semantics=("arbitrary", "arbitrary")),
    )(xyz, nx.reshape(bl * _NSB, _SBLK, 128), g1, w0a, b0v)


def _bn_affine(stv, gvec, bevec):
    s1 = jnp.sum(stv[:, 0, :], axis=0, keepdims=True)
    s2 = jnp.sum(stv[:, 1, :], axis=0, keepdims=True)
    mu = s1 / float(_NROWS)
    var = s2 / float(_NROWS) - mu * mu
    alpha = gvec * lax.rsqrt(var + _EPS)
    beta = bevec - mu * alpha
    return alpha, beta


_RBLK = 4096
_NRB = _NPOINT * _NSAMPLE // _RBLK


def _p2_body(z1_ref, st1_ref, w1t_ref, b1_ref, g1v_ref, be1_ref, st2_ref):
    al1, be1 = _bn_affine(st1_ref[...], g1v_ref[...], be1_ref[...])
    a = jnp.maximum(z1_ref[0] * al1 + be1, 0.0)
    z2 = jnp.dot(a, w1t_ref[...], preferred_element_type=_F32) + b1_ref[...]
    s1 = jnp.sum(z2, axis=0, keepdims=True)
    s2 = jnp.sum(z2 * z2, axis=0, keepdims=True)
    st2_ref[0] = jnp.concatenate(
        [s1, s2, jnp.zeros((6, _D), _F32)], axis=0)


def _run_p2(z1, st1, w1t, b1v, g1v, be1v, bl):
    nblk = bl * _NRB
    return pl.pallas_call(
        _p2_body,
        grid=(bl, _NRB),
        in_specs=[
            pl.BlockSpec((1, _RBLK, _D), lambda b, j: (b, j, 0)),
            pl.BlockSpec(st1.shape, lambda b, j: (0, 0, 0)),
            pl.BlockSpec((_D, _D), lambda b, j: (0, 0)),
            pl.BlockSpec((1, _D), lambda b, j: (0, 0)),
            pl.BlockSpec((1, _D), lambda b, j: (0, 0)),
            pl.BlockSpec((1, _D), lambda b, j: (0, 0)),
        ],
        out_specs=pl.BlockSpec((1, 8, _D), lambda b, j: (b * _NRB + j, 0, 0)),
        out_shape=jax.ShapeDtypeStruct((nblk, 8, _D), _F32),
        compiler_params=pltpu.CompilerParams(
            dimension_semantics=("arbitrary", "arbitrary")),
    )(z1, st1, w1t, b1v, g1v, be1v)


def _p3_body(z1_ref, st1_ref, st2_ref, w1t_ref, b1_ref, gl1_ref, bel1_ref,
             gl2_ref, bel2_ref, w2t_ref, b2_ref, st3_ref):
    al1, be1 = _bn_affine(st1_ref[...], gl1_ref[...], bel1_ref[...])
    a = jnp.maximum(z1_ref[0] * al1 + be1, 0.0)
    z2 = jnp.dot(a, w1t_ref[...], preferred_element_type=_F32) + b1_ref[...]
    al2, be2 = _bn_affine(st2_ref[...], gl2_ref[...], bel2_ref[...])
    a2 = jnp.maximum(z2 * al2 + be2, 0.0)
    z3 = jnp.dot(a2, w2t_ref[...], preferred_element_type=_F32) + b2_ref[...]
    s1 = jnp.sum(z3, axis=0, keepdims=True)
    s2 = jnp.sum(z3 * z3, axis=0, keepdims=True)
    st3_ref[0] = jnp.concatenate(
        [s1, s2, jnp.zeros((6, 2 * _D), _F32)], axis=0)


def _run_p3(z1, st1, st2, w1t, b1v, gl1, bel1, gl2, bel2, w2t, b2v, bl):
    nblk = bl * _NRB
    vec = lambda: pl.BlockSpec((1, _D), lambda b, j: (0, 0))
    return pl.pallas_call(
        _p3_body,
        grid=(bl, _NRB),
        in_specs=[
            pl.BlockSpec((1, _RBLK, _D), lambda b, j: (b, j, 0)),
            pl.BlockSpec(st1.shape, lambda b, j: (0, 0, 0)),
            pl.BlockSpec(st2.shape, lambda b, j: (0, 0, 0)),
            pl.BlockSpec((_D, _D), lambda b, j: (0, 0)),
            vec(), vec(), vec(), vec(), vec(),
            pl.BlockSpec((_D, 2 * _D), lambda b, j: (0, 0)),
            pl.BlockSpec((1, 2 * _D), lambda b, j: (0, 0)),
        ],
        out_specs=pl.BlockSpec((1, 8, 2 * _D),
                               lambda b, j: (b * _NRB + j, 0, 0)),
        out_shape=jax.ShapeDtypeStruct((nblk, 8, 2 * _D), _F32),
        compiler_params=pltpu.CompilerParams(
            dimension_semantics=("arbitrary", "arbitrary")),
    )(z1, st1, st2, w1t, b1v, gl1, bel1, gl2, bel2, w2t, b2v)


def _p4_body(z1_ref, st1_ref, st2_ref, st3_ref, w1t_ref, b1_ref,
             gl1_ref, bel1_ref, gl2_ref, bel2_ref, gl3_ref, bel3_ref,
             w2t_ref, b2_ref, out_ref):
    al1, be1 = _bn_affine(st1_ref[...], gl1_ref[...], bel1_ref[...])
    a = jnp.maximum(z1_ref[0] * al1 + be1, 0.0)
    z2 = jnp.dot(a, w1t_ref[...], preferred_element_type=_F32) + b1_ref[...]
    al2, be2 = _bn_affine(st2_ref[...], gl2_ref[...], bel2_ref[...])
    a2 = jnp.maximum(z2 * al2 + be2, 0.0)
    z3 = jnp.dot(a2, w2t_ref[...], preferred_element_type=_F32) + b2_ref[...]
    al3, be3 = _bn_affine(st3_ref[...], gl3_ref[...], bel3_ref[...])
    a3 = jnp.maximum(z3 * al3 + be3, 0.0)
    pooled = jnp.max(a3.reshape(_RBLK // _NSAMPLE, _NSAMPLE, 2 * _D),
                     axis=1)
    out_ref[0] = pooled


def _run_p4(z1, st1, st2, st3, w1t, b1v, gl1, bel1, gl2, bel2, gl3, bel3,
            w2t, b2v, bl):
    vec = lambda: pl.BlockSpec((1, _D), lambda b, j: (0, 0))
    vec2 = lambda: pl.BlockSpec((1, 2 * _D), lambda b, j: (0, 0))
    return pl.pallas_call(
        _p4_body,
        grid=(bl, _NRB),
        in_specs=[
            pl.BlockSpec((1, _RBLK, _D), lambda b, j: (b, j, 0)),
            pl.BlockSpec(st1.shape, lambda b, j: (0, 0, 0)),
            pl.BlockSpec(st2.shape, lambda b, j: (0, 0, 0)),
            pl.BlockSpec(st3.shape, lambda b, j: (0, 0, 0)),
            pl.BlockSpec((_D, _D), lambda b, j: (0, 0)),
            vec(), vec(), vec(), vec(), vec(),
            vec2(), vec2(),
            pl.BlockSpec((_D, 2 * _D), lambda b, j: (0, 0)),
            vec2(),
        ],
        out_specs=pl.BlockSpec((1, _RBLK // _NSAMPLE, 2 * _D),
                               lambda b, j: (b, j, 0)),
        out_shape=jax.ShapeDtypeStruct((bl, _NPOINT, 2 * _D), _F32),
        compiler_params=pltpu.CompilerParams(
            dimension_semantics=("arbitrary", "arbitrary")),
    )(z1, st1, st2, st3, w1t, b1v, gl1, bel1, gl2, bel2, gl3, bel3,
      w2t, b2v)


def _pipeline(xyz, points, w0, b0, g0, be0, w1, b1, g1, be1, w2, b2, g2,
              be2):
    bl = xyz.shape[0]
    xyz = xyz.astype(_F32)
    points = points.astype(_F32)

    nx = _run_fps(xyz, bl)

    w0a = w0[:, :3].T
    g1feat = _run_p0(xyz, points, w0a, w0[:, 3:].T, bl)

    b0v = b0.reshape(1, _D)
    z1, st1 = _run_p1(xyz, nx, g1feat, w0a, b0v, bl)
    st1 = lax.psum(st1, "c")

    gl1 = g0.reshape(1, _D)
    bel1 = be0.reshape(1, _D)
    gl2 = g1.reshape(1, _D)
    bel2 = be1.reshape(1, _D)
    gl3 = g2.reshape(1, 2 * _D)
    bel3 = be2.reshape(1, 2 * _D)
    b1v = b1.reshape(1, _D)
    b2v = b2.reshape(1, 2 * _D)
    w1t = w1.T
    w2t = w2.T

    st2 = _run_p2(z1, st1, w1t, b1v, gl1, bel1, bl)
    st2 = lax.psum(st2, "c")
    st3 = _run_p3(z1, st1, st2, w1t, b1v, gl1, bel1, gl2, bel2, w2t, b2v,
                  bl)
    st3 = lax.psum(st3, "c")
    pooled = _run_p4(z1, st1, st2, st3, w1t, b1v, gl1, bel1, gl2, bel2,
                     gl3, bel3, w2t, b2v, bl)

    new_xyz = nx[:, :, :3].transpose(0, 2, 1)
    new_points = pooled.transpose(0, 2, 1)
    return new_xyz, new_points


def kernel(xyz, points, w0, b0, g0, be0, w1, b1, g1, be1, w2, b2, g2, be2):
    devs = jax.devices()
    nd = 2 if (len(devs) >= 2 and _B % (2 * _G) == 0) else 1
    mesh = jax.sharding.Mesh(np.array(devs[:nd]), ("c",))
    shd = P("c")
    rep = P()
    f = _shard_map(
        _pipeline, mesh=mesh,
        in_specs=(shd, shd) + (rep,) * 12,
        out_specs=(shd, shd),
        check_rep=False)
    return f(xyz, points, w0, b0, g0, be0, w1, b1, g1, be1, w2, b2, g2,
             be2)
